# Initial kernel scaffold; baseline (speedup 1.0000x reference)
import jax
import jax.numpy as jnp
from jax.experimental import pallas as pl


def kernel(x, edge_index, edge_weights, W1, b1, W2, b2, Wp, bp, gamma, beta):
    raise NotImplementedError("write your pallas kernel here")


# speedup vs baseline: 4.8718x; 4.8718x over previous
import functools

import jax
import jax.numpy as jnp
from jax import lax
from jax.experimental import pallas as pl
from jax.experimental.pallas import tpu as pltpu
from jax.experimental.pallas import tpu_sc as plsc

N = 10000
E = 320000
NF = 128
NH = 8
NC = 128
ET = E + N

NUM_SC = 2
NUM_TILES = 16
NW = NUM_SC * NUM_TILES

BA = 128
BD = 32
EPW = 10368
ETP = NW * EPW

NEG = -1e30


def _k1_body(x_ref, w1a_ref, w1b_ref, b1_ref, wq_ref, uv_ref, z_ref):
    x = x_ref[...]
    u = jnp.dot(x, w1a_ref[...], preferred_element_type=jnp.float32) + b1_ref[...]
    v = jnp.dot(x, w1b_ref[...], preferred_element_type=jnp.float32)
    bn = x.shape[0]
    uv_ref[...] = jnp.concatenate(
        [u, v, jnp.zeros((bn, NF - 64), jnp.float32)], axis=1)
    z_ref[...] = jnp.dot(x, wq_ref[...], preferred_element_type=jnp.float32)


def _k1(x, w1a, w1b, b1, wq):
    bn = 1000
    grid = N // bn
    return pl.pallas_call(
        _k1_body,
        grid=(grid,),
        in_specs=[
            pl.BlockSpec((bn, NF), lambda i: (i, 0)),
            pl.BlockSpec((NF, 32), lambda i: (0, 0)),
            pl.BlockSpec((NF, 32), lambda i: (0, 0)),
            pl.BlockSpec((1, 32), lambda i: (0, 0)),
            pl.BlockSpec((NF, NH * NC), lambda i: (0, 0)),
        ],
        out_specs=[
            pl.BlockSpec((bn, NF), lambda i: (i, 0)),
            pl.BlockSpec((bn, NH * NC), lambda i: (i, 0)),
        ],
        out_shape=[
            jax.ShapeDtypeStruct((N, NF), jnp.float32),
            jax.ShapeDtypeStruct((N, NH * NC), jnp.float32),
        ],
    )(x, w1a, w1b, b1, wq)


def _sca_body(uv_hbm, src_hbm, dst_hbm, s_hbm,
              sidx, didx, ub, vb, sb, sem):
    wid = lax.axis_index("s") * NUM_SC + lax.axis_index("c")
    base = wid * EPW

    def block(i, carry):
        off = base + i * BA
        pltpu.sync_copy(src_hbm.at[pl.ds(off, BA)], sidx)
        pltpu.sync_copy(dst_hbm.at[pl.ds(off, BA)], didx)
        pltpu.async_copy(uv_hbm.at[sidx], ub, sem).wait()
        pltpu.async_copy(uv_hbm.at[didx], vb, sem).wait()

        def edge(b, c2):
            for half in range(2):
                sl = pl.ds(half * 16, 16)
                val = ub.at[b][sl] + vb.at[b][pl.ds(32 + half * 16, 16)]
                sb.at[b][sl] = jnp.maximum(val, 0.0)
            return c2

        lax.fori_loop(0, BA, edge, 0, unroll=False)
        pltpu.sync_copy(sb, s_hbm.at[pl.ds(off, BA)])
        return carry

    lax.fori_loop(0, EPW // BA, block, 0, unroll=False)


def _sca(uv, srcp, dstp):
    mesh = plsc.VectorSubcoreMesh(core_axis_name="c", subcore_axis_name="s")
    f = functools.partial(
        pl.kernel,
        mesh=mesh,
        out_type=jax.ShapeDtypeStruct((ETP, 32), jnp.float32),
        scratch_types=[
            pltpu.VMEM((BA,), jnp.int32),
            pltpu.VMEM((BA,), jnp.int32),
            pltpu.VMEM((BA, NF), jnp.float32),
            pltpu.VMEM((BA, NF), jnp.float32),
            pltpu.VMEM((BA, 32), jnp.float32),
            pltpu.SemaphoreType.DMA,
        ],
    )(_sca_body)
    return f(uv, srcp, dstp)


K2_TILE = ETP // 128


def _k2a_body(s_ref, ew_ref, w2_ref, b2_ref, t_ref, m_ref, acc):
    pid = pl.program_id(0)
    h = jnp.maximum(s_ref[...], 0.0)
    logits = jnp.dot(h, w2_ref[...], preferred_element_type=jnp.float32) + b2_ref[...]
    dyn = jax.nn.sigmoid(logits)
    t = ew_ref[...] * dyn
    t = jnp.where(t > 0, t, 0.01 * t)
    row = lax.broadcasted_iota(jnp.int32, (K2_TILE, NH), 0) + pid * K2_TILE
    t = jnp.where(row < ET, t, NEG)
    t_ref[...] = t

    @pl.when(pid == 0)
    def _():
        acc[...] = jnp.full((1, NH), NEG, jnp.float32)

    acc[...] = jnp.maximum(acc[...], jnp.max(t, axis=0, keepdims=True))

    @pl.when(pid == pl.num_programs(0) - 1)
    def _():
        m_ref[...] = acc[...]


def _k2a(s, ewt, w2, b2):
    grid = ETP // K2_TILE
    return pl.pallas_call(
        _k2a_body,
        grid=(grid,),
        in_specs=[
            pl.BlockSpec((K2_TILE, 32), lambda i: (i, 0)),
            pl.BlockSpec((K2_TILE, NH), lambda i: (i, 0)),
            pl.BlockSpec((32, NH), lambda i: (0, 0)),
            pl.BlockSpec((1, NH), lambda i: (0, 0)),
        ],
        out_specs=[
            pl.BlockSpec((K2_TILE, NH), lambda i: (i, 0)),
            pl.BlockSpec((1, NH), lambda i: (0, 0)),
        ],
        out_shape=[
            jax.ShapeDtypeStruct((ETP, NH), jnp.float32),
            jax.ShapeDtypeStruct((1, NH), jnp.float32),
        ],
        scratch_shapes=[pltpu.VMEM((1, NH), jnp.float32)],
    )(s, ewt, w2, b2)


def _k2b_body(t_ref, m_ref, w_ref, inv_ref, acc):
    pid = pl.program_id(0)
    w = jnp.exp(t_ref[...] - m_ref[...])
    w_ref[...] = w

    @pl.when(pid == 0)
    def _():
        acc[...] = jnp.zeros((1, NH), jnp.float32)

    acc[...] = acc[...] + jnp.sum(w, axis=0, keepdims=True)

    @pl.when(pid == pl.num_programs(0) - 1)
    def _():
        inv = 1.0 / acc[...]
        inv_ref[...] = jnp.concatenate([inv, inv], axis=1)


def _k2b(t, m):
    grid = ETP // K2_TILE
    return pl.pallas_call(
        _k2b_body,
        grid=(grid,),
        in_specs=[
            pl.BlockSpec((K2_TILE, NH), lambda i: (i, 0)),
            pl.BlockSpec((1, NH), lambda i: (0, 0)),
        ],
        out_specs=[
            pl.BlockSpec((K2_TILE, NH), lambda i: (i, 0)),
            pl.BlockSpec((1, 2 * NH), lambda i: (0, 0)),
        ],
        out_shape=[
            jax.ShapeDtypeStruct((ETP, NH), jnp.float32),
            jax.ShapeDtypeStruct((1, 2 * NH), jnp.float32),
        ],
        scratch_shapes=[pltpu.VMEM((1, NH), jnp.float32)],
    )(t, m)


RPT = 632
NP = NUM_TILES * RPT


def _scd_body(z_hbm, w_hbm, src_hbm, dst_hbm, inv_hbm, out_hbm,
              sidx, didx, wb, zb, cb, invv, acc, sem):
    cid = lax.axis_index("c")
    tid = lax.axis_index("s")
    base = tid * EPW + cid * (NW // 2) * EPW
    zstart = tid * RPT

    def zrow(i, carry):
        for cv in range(NC // 16):
            cb.at[i][pl.ds(cv * 16, 16)] = jnp.zeros((16,), jnp.float32)
        return carry

    lax.fori_loop(0, BD, zrow, 0, unroll=False)
    for j in range(RPT // BD):
        pltpu.sync_copy(cb, acc.at[pl.ds(zstart + j * BD, BD)])
    rem = RPT - (RPT // BD) * BD
    pltpu.sync_copy(cb.at[pl.ds(0, rem)],
                    acc.at[pl.ds(zstart + (RPT // BD) * BD, rem)])
    pltpu.sync_copy(inv_hbm, invv)
    plsc.subcore_barrier()

    def block(i, carry):
        off = base + i * BD
        pltpu.sync_copy(src_hbm.at[pl.ds(off, BD)], sidx)
        pltpu.sync_copy(dst_hbm.at[pl.ds(off, BD)], didx)
        pltpu.sync_copy(w_hbm.at[pl.ds(off * NH, BD * NH)], wb)
        pltpu.async_copy(z_hbm.at[sidx], zb, sem).wait()
        iv = invv[...]

        def wscale(j, carry2):
            wb.at[pl.ds(j * 16, 16)][...] = wb.at[pl.ds(j * 16, 16)][...] * iv
            return carry2

        lax.fori_loop(0, BD * NH // 16, wscale, 0, unroll=False)

        def pair(j, carry2):
            wvec = wb.at[pl.ds(j * 16, 16)][...]
            for par in range(2):
                b = j * 2 + par
                accs = [jnp.zeros((16,), jnp.float32) for _ in range(NC // 16)]
                for h in range(NH):
                    wbh = wvec[par * NH + h]
                    for cv in range(NC // 16):
                        seg = zb.at[b][pl.ds(h * NC + cv * 16, 16)][...]
                        accs[cv] = accs[cv] + wbh * seg
                for cv in range(NC // 16):
                    cb.at[b][pl.ds(cv * 16, 16)] = accs[cv]
            return carry2

        lax.fori_loop(0, BD // 2, pair, 0, unroll=False)
        pltpu.sync_copy(cb, acc.at[didx], add=True)
        return carry

    lax.fori_loop(0, EPW // BD, block, 0, unroll=False)
    plsc.subcore_barrier()
    pltpu.sync_copy(acc.at[pl.ds(zstart, RPT)],
                    out_hbm.at[cid, pl.ds(zstart, RPT)])


def _scd(z, wexp_flat, srcp, dstp, inv16):
    mesh = plsc.VectorSubcoreMesh(core_axis_name="c", subcore_axis_name="s")
    f = functools.partial(
        pl.kernel,
        mesh=mesh,
        out_type=jax.ShapeDtypeStruct((NUM_SC, NP, NC), jnp.float32),
        scratch_types=[
            pltpu.VMEM((BD,), jnp.int32),
            pltpu.VMEM((BD,), jnp.int32),
            pltpu.VMEM((BD * NH,), jnp.float32),
            pltpu.VMEM((BD, NH * NC), jnp.float32),
            pltpu.VMEM((BD, NC), jnp.float32),
            pltpu.VMEM((16,), jnp.float32),
            pltpu.VMEM_SHARED((NP, NC), jnp.float32),
            pltpu.SemaphoreType.DMA,
        ],
    )(_scd_body)
    return f(z, wexp_flat, srcp, dstp, inv16)


def _k3_body(p_ref, bp_ref, g_ref, bt_ref, o_ref):
    s = p_ref[0] + p_ref[1] + bp_ref[...]
    mean = jnp.mean(s, axis=-1, keepdims=True)
    d = s - mean
    var = jnp.mean(d * d, axis=-1, keepdims=True)
    o_ref[...] = d * lax.rsqrt(var + 1e-5) * g_ref[...] + bt_ref[...]


def _k3(partials, bp, gamma, beta):
    bn = 1000
    grid = N // bn
    return pl.pallas_call(
        _k3_body,
        grid=(grid,),
        in_specs=[
            pl.BlockSpec((NUM_SC, bn, NC), lambda i: (0, i, 0)),
            pl.BlockSpec((1, NC), lambda i: (0, 0)),
            pl.BlockSpec((1, NC), lambda i: (0, 0)),
            pl.BlockSpec((1, NC), lambda i: (0, 0)),
        ],
        out_specs=pl.BlockSpec((bn, NC), lambda i: (i, 0)),
        out_shape=jax.ShapeDtypeStruct((N, NC), jnp.float32),
    )(partials, bp, gamma, beta)


def kernel(x, edge_index, edge_weights, W1, b1, W2, b2, Wp, bp, gamma, beta):
    loops = jnp.arange(N, dtype=edge_index.dtype)
    src = jnp.concatenate([edge_index[0], loops,
                           jnp.zeros((ETP - ET,), edge_index.dtype)])
    dst = jnp.concatenate([edge_index[1], loops,
                           jnp.zeros((ETP - ET,), edge_index.dtype)])
    ewt = jnp.pad(edge_weights.T, ((0, ETP - ET), (0, 0)))

    w1a = W1[:NF]
    w1b = W1[NF:]
    wq = Wp.reshape(NH, NF, NC).transpose(1, 0, 2).reshape(NF, NH * NC)

    uv, z = _k1(x, w1a, w1b, b1.reshape(1, 32), wq)
    s = _sca(uv, src, dst)
    t, m = _k2a(s, ewt, W2, b2.reshape(1, NH))
    wexp, inv = _k2b(t, m)
    partials = _scd(z, wexp.reshape(ETP * NH), src, dst, inv.reshape(2 * NH))
    return _k3(partials, bp.reshape(1, NC), gamma.reshape(1, NC),
               beta.reshape(1, NC))


# speedup vs baseline: 5.3154x; 1.0911x over previous
import functools

import jax
import jax.numpy as jnp
from jax import lax
from jax.experimental import pallas as pl
from jax.experimental.pallas import tpu as pltpu
from jax.experimental.pallas import tpu_sc as plsc

N = 10000
E = 320000
NF = 128
NH = 8
NC = 128
ET = E + N

NUM_SC = 2
NUM_TILES = 16
NW = NUM_SC * NUM_TILES

BA = 128
BD = 16
GD = 8
EPW = 10368
ETP = NW * EPW

NEG = -1e30


def _k1_body(x_ref, w1a_ref, w1b_ref, b1_ref, wq_ref, uv_ref, z_ref):
    x = x_ref[...]
    u = jnp.dot(x, w1a_ref[...], preferred_element_type=jnp.float32) + b1_ref[...]
    v = jnp.dot(x, w1b_ref[...], preferred_element_type=jnp.float32)
    bn = x.shape[0]
    uv_ref[...] = jnp.concatenate(
        [u, v, jnp.zeros((bn, NF - 64), jnp.float32)], axis=1)
    z_ref[...] = jnp.dot(x, wq_ref[...], preferred_element_type=jnp.float32)


def _k1(x, w1a, w1b, b1, wq):
    bn = 1000
    grid = N // bn
    return pl.pallas_call(
        _k1_body,
        grid=(grid,),
        in_specs=[
            pl.BlockSpec((bn, NF), lambda i: (i, 0)),
            pl.BlockSpec((NF, 32), lambda i: (0, 0)),
            pl.BlockSpec((NF, 32), lambda i: (0, 0)),
            pl.BlockSpec((1, 32), lambda i: (0, 0)),
            pl.BlockSpec((NF, NH * NC), lambda i: (0, 0)),
        ],
        out_specs=[
            pl.BlockSpec((bn, NF), lambda i: (i, 0)),
            pl.BlockSpec((bn, NH * NC), lambda i: (i, 0)),
        ],
        out_shape=[
            jax.ShapeDtypeStruct((N, NF), jnp.float32),
            jax.ShapeDtypeStruct((N, NH * NC), jnp.float32),
        ],
    )(x, w1a, w1b, b1, wq)


def _sca_body(uv_hbm, src_hbm, dst_hbm, s_hbm,
              sidx, didx, ub, vb, sb, sem):
    wid = lax.axis_index("s") * NUM_SC + lax.axis_index("c")
    base = wid * EPW

    def block(i, carry):
        off = base + i * BA
        pltpu.sync_copy(src_hbm.at[pl.ds(off, BA)], sidx)
        pltpu.sync_copy(dst_hbm.at[pl.ds(off, BA)], didx)
        pltpu.async_copy(uv_hbm.at[sidx], ub, sem).wait()
        pltpu.async_copy(uv_hbm.at[didx], vb, sem).wait()

        def edge(b, c2):
            for half in range(2):
                sl = pl.ds(half * 16, 16)
                val = ub.at[b][sl] + vb.at[b][pl.ds(32 + half * 16, 16)]
                sb.at[b][sl] = jnp.maximum(val, 0.0)
            return c2

        lax.fori_loop(0, BA, edge, 0, unroll=False)
        pltpu.sync_copy(sb, s_hbm.at[pl.ds(off, BA)])
        return carry

    lax.fori_loop(0, EPW // BA, block, 0, unroll=False)


def _sca(uv, srcp, dstp):
    mesh = plsc.VectorSubcoreMesh(core_axis_name="c", subcore_axis_name="s")
    f = functools.partial(
        pl.kernel,
        mesh=mesh,
        out_type=jax.ShapeDtypeStruct((ETP, 32), jnp.float32),
        scratch_types=[
            pltpu.VMEM((BA,), jnp.int32),
            pltpu.VMEM((BA,), jnp.int32),
            pltpu.VMEM((BA, NF), jnp.float32),
            pltpu.VMEM((BA, NF), jnp.float32),
            pltpu.VMEM((BA, 32), jnp.float32),
            pltpu.SemaphoreType.DMA,
        ],
    )(_sca_body)
    return f(uv, srcp, dstp)


K2_TILE = ETP // 128


def _k2a_body(s_ref, ew_ref, w2_ref, b2_ref, t_ref, m_ref, acc):
    pid = pl.program_id(0)
    h = jnp.maximum(s_ref[...], 0.0)
    logits = jnp.dot(h, w2_ref[...], preferred_element_type=jnp.float32) + b2_ref[...]
    dyn = jax.nn.sigmoid(logits)
    t = ew_ref[...] * dyn
    t = jnp.where(t > 0, t, 0.01 * t)
    row = lax.broadcasted_iota(jnp.int32, (K2_TILE, NH), 0) + pid * K2_TILE
    t = jnp.where(row < ET, t, NEG)
    t_ref[...] = t

    @pl.when(pid == 0)
    def _():
        acc[...] = jnp.full((1, NH), NEG, jnp.float32)

    acc[...] = jnp.maximum(acc[...], jnp.max(t, axis=0, keepdims=True))

    @pl.when(pid == pl.num_programs(0) - 1)
    def _():
        m_ref[...] = acc[...]


def _k2a(s, ewt, w2, b2):
    grid = ETP // K2_TILE
    return pl.pallas_call(
        _k2a_body,
        grid=(grid,),
        in_specs=[
            pl.BlockSpec((K2_TILE, 32), lambda i: (i, 0)),
            pl.BlockSpec((K2_TILE, NH), lambda i: (i, 0)),
            pl.BlockSpec((32, NH), lambda i: (0, 0)),
            pl.BlockSpec((1, NH), lambda i: (0, 0)),
        ],
        out_specs=[
            pl.BlockSpec((K2_TILE, NH), lambda i: (i, 0)),
            pl.BlockSpec((1, NH), lambda i: (0, 0)),
        ],
        out_shape=[
            jax.ShapeDtypeStruct((ETP, NH), jnp.float32),
            jax.ShapeDtypeStruct((1, NH), jnp.float32),
        ],
        scratch_shapes=[pltpu.VMEM((1, NH), jnp.float32)],
    )(s, ewt, w2, b2)


def _k2b_body(t_ref, m_ref, w_ref, inv_ref, acc):
    pid = pl.program_id(0)
    w = jnp.exp(t_ref[...] - m_ref[...])
    w_ref[...] = w

    @pl.when(pid == 0)
    def _():
        acc[...] = jnp.zeros((1, NH), jnp.float32)

    acc[...] = acc[...] + jnp.sum(w, axis=0, keepdims=True)

    @pl.when(pid == pl.num_programs(0) - 1)
    def _():
        inv = 1.0 / acc[...]
        inv_ref[...] = jnp.concatenate([inv, inv], axis=1)


def _k2b(t, m):
    grid = ETP // K2_TILE
    return pl.pallas_call(
        _k2b_body,
        grid=(grid,),
        in_specs=[
            pl.BlockSpec((K2_TILE, NH), lambda i: (i, 0)),
            pl.BlockSpec((1, NH), lambda i: (0, 0)),
        ],
        out_specs=[
            pl.BlockSpec((K2_TILE, NH), lambda i: (i, 0)),
            pl.BlockSpec((1, 2 * NH), lambda i: (0, 0)),
        ],
        out_shape=[
            jax.ShapeDtypeStruct((ETP, NH), jnp.float32),
            jax.ShapeDtypeStruct((1, 2 * NH), jnp.float32),
        ],
        scratch_shapes=[pltpu.VMEM((1, NH), jnp.float32)],
    )(t, m)


RPT = 632
NP = NUM_TILES * RPT


NB_D = EPW // BD
NG_D = NB_D // GD


def _scd_body(z_hbm, w_hbm, src_hbm, dst_hbm, inv_hbm, out_hbm,
              sidx, didxg, wbg, zba, zbb, cb, invv, acc, sema, semb):
    cid = lax.axis_index("c")
    tid = lax.axis_index("s")
    base = tid * EPW + cid * (NW // 2) * EPW
    zstart = tid * RPT

    def zrow(i, carry):
        for cv in range(NC // 16):
            cb.at[i][pl.ds(cv * 16, 16)] = jnp.zeros((16,), jnp.float32)
        return carry

    lax.fori_loop(0, BD, zrow, 0, unroll=False)
    for j in range(RPT // BD):
        pltpu.sync_copy(cb, acc.at[pl.ds(zstart + j * BD, BD)])
    rem = RPT - (RPT // BD) * BD
    pltpu.sync_copy(cb.at[pl.ds(0, rem)],
                    acc.at[pl.ds(zstart + (RPT // BD) * BD, rem)])
    pltpu.sync_copy(inv_hbm, invv)
    pltpu.sync_copy(src_hbm.at[pl.ds(base, EPW)], sidx)
    plsc.subcore_barrier()

    def issue(bi, zb, sem):
        bc = jnp.minimum(bi, NB_D - 1)
        pltpu.async_copy(z_hbm.at[sidx.at[pl.ds(bc * BD, BD)]], zb, sem)

    def drain(zb, sem):
        pltpu.make_async_copy(z_hbm.at[pl.ds(0, BD)], zb, sem).wait()

    def grp_load(g):
        off = base + g * (GD * BD)
        pltpu.sync_copy(dst_hbm.at[pl.ds(off, GD * BD)], didxg)
        pltpu.sync_copy(w_hbm.at[pl.ds(off * NH, GD * BD * NH)], wbg)
        iv = invv[...]

        def wsc(k, c):
            wbg.at[pl.ds(k * 16, 16)][...] = wbg.at[pl.ds(k * 16, 16)][...] * iv
            return c

        lax.fori_loop(0, GD * BD * NH // 16, wsc, 0, unroll=False)

    def compute(bi, zb):
        slot = lax.rem(bi, GD)
        for p in range(BD // 2):
            wvec = wbg[pl.ds((slot * BD + p * 2) * NH, 16)]
            for par in range(2):
                b = p * 2 + par
                accs = [jnp.zeros((16,), jnp.float32) for _ in range(NC // 16)]
                for h in range(NH):
                    wbh = wvec[par * NH + h]
                    for cv in range(NC // 16):
                        seg = zb.at[b][pl.ds(h * NC + cv * 16, 16)][...]
                        accs[cv] = accs[cv] + wbh * seg
                for cv in range(NC // 16):
                    cb.at[b][pl.ds(cv * 16, 16)] = accs[cv]
        pltpu.sync_copy(cb, acc.at[didxg.at[pl.ds(slot * BD, BD)]], add=True)

    issue(0, zba, sema)
    issue(1, zbb, semb)

    def body(j, carry):
        bx = 2 * j

        @pl.when(lax.rem(bx, GD) == 0)
        def _():
            grp_load(bx // GD)

        drain(zba, sema)
        compute(bx, zba)
        issue(bx + 2, zba, sema)
        drain(zbb, semb)
        compute(bx + 1, zbb)
        issue(bx + 3, zbb, semb)
        return carry

    lax.fori_loop(0, NB_D // 2, body, 0, unroll=False)
    drain(zba, sema)
    drain(zbb, semb)
    plsc.subcore_barrier()
    pltpu.sync_copy(acc.at[pl.ds(zstart, RPT)],
                    out_hbm.at[cid, pl.ds(zstart, RPT)])


def _scd(z, wexp_flat, srcp, dstp, inv16):
    mesh = plsc.VectorSubcoreMesh(core_axis_name="c", subcore_axis_name="s")
    f = functools.partial(
        pl.kernel,
        mesh=mesh,
        out_type=jax.ShapeDtypeStruct((NUM_SC, NP, NC), jnp.float32),
        scratch_types=[
            pltpu.VMEM((EPW,), jnp.int32),
            pltpu.VMEM((GD * BD,), jnp.int32),
            pltpu.VMEM((GD * BD * NH,), jnp.float32),
            pltpu.VMEM((BD, NH * NC), jnp.float32),
            pltpu.VMEM((BD, NH * NC), jnp.float32),
            pltpu.VMEM((BD, NC), jnp.float32),
            pltpu.VMEM((16,), jnp.float32),
            pltpu.VMEM_SHARED((NP, NC), jnp.float32),
            pltpu.SemaphoreType.DMA,
            pltpu.SemaphoreType.DMA,
        ],
    )(_scd_body)
    return f(z, wexp_flat, srcp, dstp, inv16)


def _k3_body(p_ref, bp_ref, g_ref, bt_ref, o_ref):
    s = p_ref[0] + p_ref[1] + bp_ref[...]
    mean = jnp.mean(s, axis=-1, keepdims=True)
    d = s - mean
    var = jnp.mean(d * d, axis=-1, keepdims=True)
    o_ref[...] = d * lax.rsqrt(var + 1e-5) * g_ref[...] + bt_ref[...]


def _k3(partials, bp, gamma, beta):
    bn = 1000
    grid = N // bn
    return pl.pallas_call(
        _k3_body,
        grid=(grid,),
        in_specs=[
            pl.BlockSpec((NUM_SC, bn, NC), lambda i: (0, i, 0)),
            pl.BlockSpec((1, NC), lambda i: (0, 0)),
            pl.BlockSpec((1, NC), lambda i: (0, 0)),
            pl.BlockSpec((1, NC), lambda i: (0, 0)),
        ],
        out_specs=pl.BlockSpec((bn, NC), lambda i: (i, 0)),
        out_shape=jax.ShapeDtypeStruct((N, NC), jnp.float32),
    )(partials, bp, gamma, beta)


def kernel(x, edge_index, edge_weights, W1, b1, W2, b2, Wp, bp, gamma, beta):
    loops = jnp.arange(N, dtype=edge_index.dtype)
    src = jnp.concatenate([edge_index[0], loops,
                           jnp.zeros((ETP - ET,), edge_index.dtype)])
    dst = jnp.concatenate([edge_index[1], loops,
                           jnp.zeros((ETP - ET,), edge_index.dtype)])
    ewt = jnp.pad(edge_weights.T, ((0, ETP - ET), (0, 0)))

    w1a = W1[:NF]
    w1b = W1[NF:]
    wq = Wp.reshape(NH, NF, NC).transpose(1, 0, 2).reshape(NF, NH * NC)

    uv, z = _k1(x, w1a, w1b, b1.reshape(1, 32), wq)
    s = _sca(uv, src, dst)
    t, m = _k2a(s, ewt, W2, b2.reshape(1, NH))
    wexp, inv = _k2b(t, m)
    partials = _scd(z, wexp.reshape(ETP * NH), src, dst, inv.reshape(2 * NH))
    return _k3(partials, bp.reshape(1, NC), gamma.reshape(1, NC),
               beta.reshape(1, NC))


# speedup vs baseline: 5.3464x; 1.0058x over previous
import functools

import jax
import jax.numpy as jnp
from jax import lax
from jax.experimental import pallas as pl
from jax.experimental.pallas import tpu as pltpu
from jax.experimental.pallas import tpu_sc as plsc

N = 10000
E = 320000
NF = 128
NH = 8
NC = 128
ET = E + N

NUM_SC = 2
NUM_TILES = 16
NW = NUM_SC * NUM_TILES

BA = 128
BD = 16
GD = 8
EPW = 10368
ETP = NW * EPW

NEG = -1e30


def _k1_body(x_ref, w1a_ref, w1b_ref, b1_ref, wq_ref, uv_ref, z_ref):
    x = x_ref[...]
    u = jnp.dot(x, w1a_ref[...], preferred_element_type=jnp.float32) + b1_ref[...]
    v = jnp.dot(x, w1b_ref[...], preferred_element_type=jnp.float32)
    bn = x.shape[0]
    uv_ref[...] = jnp.concatenate(
        [u, v, jnp.zeros((bn, NF - 64), jnp.float32)], axis=1)
    z_ref[...] = jnp.dot(x, wq_ref[...], preferred_element_type=jnp.float32)


def _k1(x, w1a, w1b, b1, wq):
    bn = 2000
    grid = N // bn
    return pl.pallas_call(
        _k1_body,
        grid=(grid,),
        in_specs=[
            pl.BlockSpec((bn, NF), lambda i: (i, 0)),
            pl.BlockSpec((NF, 32), lambda i: (0, 0)),
            pl.BlockSpec((NF, 32), lambda i: (0, 0)),
            pl.BlockSpec((1, 32), lambda i: (0, 0)),
            pl.BlockSpec((NF, NH * NC), lambda i: (0, 0)),
        ],
        out_specs=[
            pl.BlockSpec((bn, NF), lambda i: (i, 0)),
            pl.BlockSpec((bn, NH * NC), lambda i: (i, 0)),
        ],
        out_shape=[
            jax.ShapeDtypeStruct((N, NF), jnp.float32),
            jax.ShapeDtypeStruct((N, NH * NC), jnp.float32),
        ],
    )(x, w1a, w1b, b1, wq)


def _sca_body(uv_hbm, src_hbm, dst_hbm, s_hbm,
              sidx, didx, ub, vb, sb, sem):
    wid = lax.axis_index("s") * NUM_SC + lax.axis_index("c")
    base = wid * EPW

    def block(i, carry):
        off = base + i * BA
        pltpu.sync_copy(src_hbm.at[pl.ds(off, BA)], sidx)
        pltpu.sync_copy(dst_hbm.at[pl.ds(off, BA)], didx)
        pltpu.async_copy(uv_hbm.at[sidx], ub, sem).wait()
        pltpu.async_copy(uv_hbm.at[didx], vb, sem).wait()

        def edge(b, c2):
            for half in range(2):
                sl = pl.ds(half * 16, 16)
                val = ub.at[b][sl] + vb.at[b][pl.ds(32 + half * 16, 16)]
                sb.at[b][sl] = jnp.maximum(val, 0.0)
            return c2

        lax.fori_loop(0, BA, edge, 0, unroll=False)
        pltpu.sync_copy(sb, s_hbm.at[pl.ds(off, BA)])
        return carry

    lax.fori_loop(0, EPW // BA, block, 0, unroll=False)


def _sca(uv, srcp, dstp):
    mesh = plsc.VectorSubcoreMesh(core_axis_name="c", subcore_axis_name="s")
    f = functools.partial(
        pl.kernel,
        mesh=mesh,
        out_type=jax.ShapeDtypeStruct((ETP, 32), jnp.float32),
        scratch_types=[
            pltpu.VMEM((BA,), jnp.int32),
            pltpu.VMEM((BA,), jnp.int32),
            pltpu.VMEM((BA, NF), jnp.float32),
            pltpu.VMEM((BA, NF), jnp.float32),
            pltpu.VMEM((BA, 32), jnp.float32),
            pltpu.SemaphoreType.DMA,
        ],
    )(_sca_body)
    return f(uv, srcp, dstp)


K2_TILE = ETP // 128


def _k2a_body(s_ref, ew_ref, w2_ref, b2_ref, t_ref, m_ref, acc):
    pid = pl.program_id(0)
    h = jnp.maximum(s_ref[...], 0.0)
    logits = jnp.dot(h, w2_ref[...], preferred_element_type=jnp.float32) + b2_ref[...]
    dyn = jax.nn.sigmoid(logits)
    t = ew_ref[...] * dyn
    t = jnp.where(t > 0, t, 0.01 * t)
    row = lax.broadcasted_iota(jnp.int32, (K2_TILE, NH), 0) + pid * K2_TILE
    t = jnp.where(row < ET, t, NEG)
    t_ref[...] = t

    @pl.when(pid == 0)
    def _():
        acc[...] = jnp.full((1, NH), NEG, jnp.float32)

    acc[...] = jnp.maximum(acc[...], jnp.max(t, axis=0, keepdims=True))

    @pl.when(pid == pl.num_programs(0) - 1)
    def _():
        m_ref[...] = acc[...]


def _k2a(s, ewt, w2, b2):
    grid = ETP // K2_TILE
    return pl.pallas_call(
        _k2a_body,
        grid=(grid,),
        in_specs=[
            pl.BlockSpec((K2_TILE, 32), lambda i: (i, 0)),
            pl.BlockSpec((K2_TILE, NH), lambda i: (i, 0)),
            pl.BlockSpec((32, NH), lambda i: (0, 0)),
            pl.BlockSpec((1, NH), lambda i: (0, 0)),
        ],
        out_specs=[
            pl.BlockSpec((K2_TILE, NH), lambda i: (i, 0)),
            pl.BlockSpec((1, NH), lambda i: (0, 0)),
        ],
        out_shape=[
            jax.ShapeDtypeStruct((ETP, NH), jnp.float32),
            jax.ShapeDtypeStruct((1, NH), jnp.float32),
        ],
        scratch_shapes=[pltpu.VMEM((1, NH), jnp.float32)],
    )(s, ewt, w2, b2)


def _k2b_body(t_ref, m_ref, w_ref, inv_ref, acc):
    pid = pl.program_id(0)
    w = jnp.exp(t_ref[...] - m_ref[...])
    w_ref[...] = w

    @pl.when(pid == 0)
    def _():
        acc[...] = jnp.zeros((1, NH), jnp.float32)

    acc[...] = acc[...] + jnp.sum(w, axis=0, keepdims=True)

    @pl.when(pid == pl.num_programs(0) - 1)
    def _():
        inv = 1.0 / acc[...]
        inv_ref[...] = jnp.concatenate([inv, inv], axis=1)


def _k2b(t, m):
    grid = ETP // K2_TILE
    return pl.pallas_call(
        _k2b_body,
        grid=(grid,),
        in_specs=[
            pl.BlockSpec((K2_TILE, NH), lambda i: (i, 0)),
            pl.BlockSpec((1, NH), lambda i: (0, 0)),
        ],
        out_specs=[
            pl.BlockSpec((K2_TILE, NH), lambda i: (i, 0)),
            pl.BlockSpec((1, 2 * NH), lambda i: (0, 0)),
        ],
        out_shape=[
            jax.ShapeDtypeStruct((ETP, NH), jnp.float32),
            jax.ShapeDtypeStruct((1, 2 * NH), jnp.float32),
        ],
        scratch_shapes=[pltpu.VMEM((1, NH), jnp.float32)],
    )(t, m)


RPT = 632
NP = NUM_TILES * RPT


NB_D = EPW // BD
NG_D = NB_D // GD


def _scd_body(z_hbm, w_hbm, src_hbm, dst_hbm, inv_hbm, out_hbm,
              sidx, didxg, wbg, zba, zbb, cb, invv, acc, sema, semb):
    cid = lax.axis_index("c")
    tid = lax.axis_index("s")
    base = tid * EPW + cid * (NW // 2) * EPW
    zstart = tid * RPT

    def zrow(i, carry):
        for cv in range(NC // 16):
            cb.at[i][pl.ds(cv * 16, 16)] = jnp.zeros((16,), jnp.float32)
        return carry

    lax.fori_loop(0, BD, zrow, 0, unroll=False)
    for j in range(RPT // BD):
        pltpu.sync_copy(cb, acc.at[pl.ds(zstart + j * BD, BD)])
    rem = RPT - (RPT // BD) * BD
    pltpu.sync_copy(cb.at[pl.ds(0, rem)],
                    acc.at[pl.ds(zstart + (RPT // BD) * BD, rem)])
    pltpu.sync_copy(inv_hbm, invv)
    pltpu.sync_copy(src_hbm.at[pl.ds(base, EPW)], sidx)
    plsc.subcore_barrier()

    def issue(bi, zb, sem):
        bc = jnp.minimum(bi, NB_D - 1)
        pltpu.async_copy(z_hbm.at[sidx.at[pl.ds(bc * BD, BD)]], zb, sem)

    def drain(zb, sem):
        pltpu.make_async_copy(z_hbm.at[pl.ds(0, BD)], zb, sem).wait()

    def grp_load(g):
        off = base + g * (GD * BD)
        pltpu.sync_copy(dst_hbm.at[pl.ds(off, GD * BD)], didxg)
        pltpu.sync_copy(w_hbm.at[pl.ds(off * NH, GD * BD * NH)], wbg)
        iv = invv[...]

        def wsc(k, c):
            wbg.at[pl.ds(k * 16, 16)][...] = wbg.at[pl.ds(k * 16, 16)][...] * iv
            return c

        lax.fori_loop(0, GD * BD * NH // 16, wsc, 0, unroll=False)

    def compute(bi, zb):
        slot = lax.rem(bi, GD)
        for p in range(BD // 2):
            wvec = wbg[pl.ds((slot * BD + p * 2) * NH, 16)]
            for par in range(2):
                b = p * 2 + par
                accs = [jnp.zeros((16,), jnp.float32) for _ in range(NC // 16)]
                for h in range(NH):
                    wbh = wvec[par * NH + h]
                    for cv in range(NC // 16):
                        seg = zb.at[b][pl.ds(h * NC + cv * 16, 16)][...]
                        accs[cv] = accs[cv] + wbh * seg
                for cv in range(NC // 16):
                    cb.at[b][pl.ds(cv * 16, 16)] = accs[cv]
        pltpu.sync_copy(cb, acc.at[didxg.at[pl.ds(slot * BD, BD)]], add=True)

    issue(0, zba, sema)
    issue(1, zbb, semb)

    def body(j, carry):
        bx = 2 * j

        @pl.when(lax.rem(bx, GD) == 0)
        def _():
            grp_load(bx // GD)

        drain(zba, sema)
        compute(bx, zba)
        issue(bx + 2, zba, sema)
        drain(zbb, semb)
        compute(bx + 1, zbb)
        issue(bx + 3, zbb, semb)
        return carry

    lax.fori_loop(0, NB_D // 2, body, 0, unroll=False)
    drain(zba, sema)
    drain(zbb, semb)
    plsc.subcore_barrier()
    pltpu.sync_copy(acc.at[pl.ds(zstart, RPT)],
                    out_hbm.at[cid, pl.ds(zstart, RPT)])


def _scd(z, wexp_flat, srcp, dstp, inv16):
    mesh = plsc.VectorSubcoreMesh(core_axis_name="c", subcore_axis_name="s")
    f = functools.partial(
        pl.kernel,
        mesh=mesh,
        out_type=jax.ShapeDtypeStruct((NUM_SC, NP, NC), jnp.float32),
        scratch_types=[
            pltpu.VMEM((EPW,), jnp.int32),
            pltpu.VMEM((GD * BD,), jnp.int32),
            pltpu.VMEM((GD * BD * NH,), jnp.float32),
            pltpu.VMEM((BD, NH * NC), jnp.float32),
            pltpu.VMEM((BD, NH * NC), jnp.float32),
            pltpu.VMEM((BD, NC), jnp.float32),
            pltpu.VMEM((16,), jnp.float32),
            pltpu.VMEM_SHARED((NP, NC), jnp.float32),
            pltpu.SemaphoreType.DMA,
            pltpu.SemaphoreType.DMA,
        ],
    )(_scd_body)
    return f(z, wexp_flat, srcp, dstp, inv16)


def _k3_body(p_ref, bp_ref, g_ref, bt_ref, o_ref):
    s = p_ref[0] + p_ref[1] + bp_ref[...]
    mean = jnp.mean(s, axis=-1, keepdims=True)
    d = s - mean
    var = jnp.mean(d * d, axis=-1, keepdims=True)
    o_ref[...] = d * lax.rsqrt(var + 1e-5) * g_ref[...] + bt_ref[...]


def _k3(partials, bp, gamma, beta):
    bn = 1000
    grid = N // bn
    return pl.pallas_call(
        _k3_body,
        grid=(grid,),
        in_specs=[
            pl.BlockSpec((NUM_SC, bn, NC), lambda i: (0, i, 0)),
            pl.BlockSpec((1, NC), lambda i: (0, 0)),
            pl.BlockSpec((1, NC), lambda i: (0, 0)),
            pl.BlockSpec((1, NC), lambda i: (0, 0)),
        ],
        out_specs=pl.BlockSpec((bn, NC), lambda i: (i, 0)),
        out_shape=jax.ShapeDtypeStruct((N, NC), jnp.float32),
    )(partials, bp, gamma, beta)


def kernel(x, edge_index, edge_weights, W1, b1, W2, b2, Wp, bp, gamma, beta):
    loops = jnp.arange(N, dtype=edge_index.dtype)
    src = jnp.concatenate([edge_index[0], loops,
                           jnp.zeros((ETP - ET,), edge_index.dtype)])
    dst = jnp.concatenate([edge_index[1], loops,
                           jnp.zeros((ETP - ET,), edge_index.dtype)])
    ewt = jnp.pad(edge_weights.T, ((0, ETP - ET), (0, 0)))

    w1a = W1[:NF]
    w1b = W1[NF:]
    wq = Wp.reshape(NH, NF, NC).transpose(1, 0, 2).reshape(NF, NH * NC)

    uv, z = _k1(x, w1a, w1b, b1.reshape(1, 32), wq)
    s = _sca(uv, src, dst)
    t, m = _k2a(s, ewt, W2, b2.reshape(1, NH))
    wexp, inv = _k2b(t, m)
    partials = _scd(z, wexp.reshape(ETP * NH), src, dst, inv.reshape(2 * NH))
    return _k3(partials, bp.reshape(1, NC), gamma.reshape(1, NC),
               beta.reshape(1, NC))
